# Initial kernel scaffold; baseline (speedup 1.0000x reference)
import jax
import jax.numpy as jnp
from jax.experimental import pallas as pl


def kernel(atomic_embedding, f_ij, Wg, Wi, bi, Wj, bj, Wv, bv, gate, res_W1, res_b1, res_W2, res_b2, ores_W1, ores_b1, ores_W2, ores_b2, Wo, bo, pair_indices):
    raise NotImplementedError("write your pallas kernel here")


# speedup vs baseline: 2.2343x; 2.2343x over previous
import functools

import jax
import jax.numpy as jnp
from jax import lax
from jax.experimental import pallas as pl
from jax.experimental.pallas import tpu as pltpu
from jax.experimental.pallas import tpu_sc as plsc

N = 10000
F = 128
E = 320000

NC = 2
NS = 16
NW = NC * NS
EDGES_PER_W = E // NW
CHUNK = 80
NCHUNK = EDGES_PER_W // CHUNK
N_PAD = 10240
ROWS_PER_TILE = N_PAD // NS


def _softplus(x):
    return jnp.maximum(x, 0.0) + jnp.log1p(jnp.exp(-jnp.abs(x)))


def _node_pre_body(a_ref, wi_ref, bi_ref, wj_ref, bj_ref,
                   emb_ref, x0_ref, t_ref):
    e = _softplus(a_ref[...])
    emb_ref[...] = e
    x0_ref[...] = _softplus(
        jnp.dot(e, wi_ref[...], preferred_element_type=jnp.float32) + bi_ref[...])
    t_ref[...] = _softplus(
        jnp.dot(e, wj_ref[...], preferred_element_type=jnp.float32) + bj_ref[...])


def _node_pre(a, Wi, bi, Wj, bj):
    nb = 1000
    grid = N // nb
    blk = pl.BlockSpec((nb, F), lambda i: (i, 0))
    wblk = pl.BlockSpec((F, F), lambda i: (0, 0))
    bblk = pl.BlockSpec((1, F), lambda i: (0, 0))
    out = jax.ShapeDtypeStruct((N, F), jnp.float32)
    return pl.pallas_call(
        _node_pre_body,
        grid=(grid,),
        in_specs=[blk, wblk, bblk, wblk, bblk],
        out_specs=[blk, blk, blk],
        out_shape=[out, out, out],
    )(a, Wi, bi.reshape(1, F), Wj, bj.reshape(1, F))


def _edge_g_body(f_ref, wg_ref, g_ref):
    g_ref[...] = jnp.dot(f_ref[...], wg_ref[...],
                         preferred_element_type=jnp.float32)


def _edge_g(f_ij, Wg):
    eb = 2000
    grid = E // eb
    return pl.pallas_call(
        _edge_g_body,
        grid=(grid,),
        in_specs=[pl.BlockSpec((eb, 16), lambda i: (i, 0)),
                  pl.BlockSpec((16, F), lambda i: (0, 0))],
        out_specs=pl.BlockSpec((eb, F), lambda i: (i, 0)),
        out_shape=jax.ShapeDtypeStruct((E, F), jnp.float32),
    )(f_ij, Wg)


def _sc_edges_body(t_hbm, g_hbm, idxi_hbm, idxj_hbm, out_hbm,
                   idxj_v, idxi_v, g_v, rows_v, sem, acc_sh):
    c = lax.axis_index("c")
    s = lax.axis_index("s")
    wid = c * NS + s

    def _zrow(r, _):
        for cc in range(F // 16):
            rows_v[r, pl.ds(cc * 16, 16)] = jnp.zeros((16,), jnp.float32)
        return 0
    lax.fori_loop(0, CHUNK, _zrow, 0)

    base_row = s * ROWS_PER_TILE
    for z in range(ROWS_PER_TILE // CHUNK):
        pltpu.sync_copy(rows_v, acc_sh.at[pl.ds(base_row + z * CHUNK, CHUNK)])
    plsc.subcore_barrier()

    ebase = wid * EDGES_PER_W

    def _chunk(k, _):
        b = ebase + k * CHUNK
        pltpu.sync_copy(idxj_hbm.at[pl.ds(b, CHUNK)], idxj_v)
        pltpu.sync_copy(idxi_hbm.at[pl.ds(b, CHUNK)], idxi_v)
        pltpu.sync_copy(g_hbm.at[pl.ds(b, CHUNK)], g_v)
        pltpu.async_copy(t_hbm.at[idxj_v], rows_v, sem).wait()

        def _mul(r, _):
            for cc in range(F // 16):
                sl = pl.ds(cc * 16, 16)
                rows_v[r, sl] = rows_v[r, sl] * g_v[r, sl]
            return 0
        lax.fori_loop(0, CHUNK, _mul, 0)

        pltpu.sync_copy(rows_v, acc_sh.at[idxi_v], add=True)
        return 0

    lax.fori_loop(0, NCHUNK, _chunk, 0)
    plsc.subcore_barrier()

    pltpu.sync_copy(acc_sh.at[pl.ds(base_row, ROWS_PER_TILE)],
                    out_hbm.at[c, pl.ds(base_row, ROWS_PER_TILE)])


def _sc_edges(t, g, pair_indices):
    mesh = plsc.VectorSubcoreMesh(core_axis_name="c", subcore_axis_name="s")
    return pl.kernel(
        _sc_edges_body,
        out_type=jax.ShapeDtypeStruct((NC, N_PAD, F), jnp.float32),
        mesh=mesh,
        scratch_types=[
            pltpu.VMEM((CHUNK,), jnp.int32),
            pltpu.VMEM((CHUNK,), jnp.int32),
            pltpu.VMEM((CHUNK, F), jnp.float32),
            pltpu.VMEM((CHUNK, F), jnp.float32),
            pltpu.SemaphoreType.DMA,
            pltpu.VMEM_SHARED((N_PAD, F), jnp.float32),
        ],
    )(t, g, pair_indices[0], pair_indices[1])


def _post_body(x0_ref, p_ref, emb_ref, rw1_ref, rb1_ref, rw2_ref, rb2_ref,
               gate_ref, wv_ref, bv_ref, ow1_ref, ob1_ref, ow2_ref, ob2_ref,
               wo_ref, bo_ref, eo_ref, pred_ref):
    x = x0_ref[...] + p_ref[0] + p_ref[1]
    for r in range(3):
        h = _softplus(x)
        h = _softplus(jnp.dot(h, rw1_ref[r], preferred_element_type=jnp.float32)
                      + rb1_ref[r])
        h = jnp.dot(h, rw2_ref[r], preferred_element_type=jnp.float32) + rb2_ref[r]
        x = x + h
    x = _softplus(x)
    eo = gate_ref[...] * emb_ref[...] + (
        jnp.dot(x, wv_ref[...], preferred_element_type=jnp.float32) + bv_ref[...])
    eo_ref[...] = eo
    y = eo
    for r in range(2):
        h = _softplus(y)
        h = _softplus(jnp.dot(h, ow1_ref[r], preferred_element_type=jnp.float32)
                      + ob1_ref[r])
        h = jnp.dot(h, ow2_ref[r], preferred_element_type=jnp.float32) + ob2_ref[r]
        y = y + h
    pred_ref[...] = jnp.dot(y, wo_ref[...],
                            preferred_element_type=jnp.float32) + bo_ref[...]


def _post(x0, partials, emb, res_W1, res_b1, res_W2, res_b2, gate, Wv, bv,
          ores_W1, ores_b1, ores_W2, ores_b2, Wo_pad, bo_pad):
    nb = 1000
    grid = N // nb
    blk = pl.BlockSpec((nb, F), lambda i: (i, 0))
    pblk = pl.BlockSpec((NC, nb, F), lambda i: (0, i, 0))
    wfull = pl.BlockSpec((F, F), lambda i: (0, 0))
    brow = pl.BlockSpec((1, F), lambda i: (0, 0))
    w3 = pl.BlockSpec((3, F, F), lambda i: (0, 0, 0))
    b3 = pl.BlockSpec((3, 1, F), lambda i: (0, 0, 0))
    w2 = pl.BlockSpec((2, F, F), lambda i: (0, 0, 0))
    b2 = pl.BlockSpec((2, 1, F), lambda i: (0, 0, 0))
    out = jax.ShapeDtypeStruct((N, F), jnp.float32)
    return pl.pallas_call(
        _post_body,
        grid=(grid,),
        in_specs=[blk, pblk, blk, w3, b3, w3, b3, brow, wfull, brow,
                  w2, b2, w2, b2, wfull, brow],
        out_specs=[blk, blk],
        out_shape=[out, out],
    )(x0, partials, emb,
      res_W1, res_b1.reshape(3, 1, F), res_W2, res_b2.reshape(3, 1, F),
      gate.reshape(1, F), Wv, bv.reshape(1, F),
      ores_W1, ores_b1.reshape(2, 1, F), ores_W2, ores_b2.reshape(2, 1, F),
      Wo_pad, bo_pad)


def kernel(atomic_embedding, f_ij, Wg, Wi, bi, Wj, bj, Wv, bv, gate,
           res_W1, res_b1, res_W2, res_b2,
           ores_W1, ores_b1, ores_W2, ores_b2, Wo, bo, pair_indices):
    emb, x0, t = _node_pre(atomic_embedding, Wi, bi, Wj, bj)
    g = _edge_g(f_ij, Wg)
    partials = _sc_edges(t, g, pair_indices)[:, :N, :]
    Wo_pad = jnp.zeros((F, F), jnp.float32).at[:, :2].set(Wo)
    bo_pad = jnp.zeros((1, F), jnp.float32).at[:, :2].set(bo.reshape(1, 2))
    emb_out, pred_pad = _post(x0, partials, emb,
                              res_W1, res_b1, res_W2, res_b2, gate, Wv, bv,
                              ores_W1, ores_b1, ores_W2, ores_b2, Wo_pad, bo_pad)
    return (pred_pad[:, :2], emb_out)


# speedup vs baseline: 2.4311x; 1.0881x over previous
import functools

import jax
import jax.numpy as jnp
from jax import lax
from jax.experimental import pallas as pl
from jax.experimental.pallas import tpu as pltpu
from jax.experimental.pallas import tpu_sc as plsc

N = 10000
F = 128
E = 320000

NC = 2
NS = 16
NW = NC * NS
EDGES_PER_W = E // NW
CHUNK = 64
PAIRS = 79
CP_PROC = 2 * PAIRS
CP_RGN = CP_PROC + 2
RGN = CP_RGN * CHUNK
E_PAD = NW * RGN
PAD_ROW = 10200
N_PAD = 10240
ROWS_PER_TILE = N_PAD // NS


def _softplus(x):
    return jnp.maximum(x, 0.0) + jnp.log1p(jnp.exp(-jnp.abs(x)))


def _node_pre_body(a_ref, wi_ref, bi_ref, wj_ref, bj_ref,
                   emb_ref, x0_ref, t_ref):
    e = _softplus(a_ref[...])
    emb_ref[...] = e
    x0_ref[...] = _softplus(
        jnp.dot(e, wi_ref[...], preferred_element_type=jnp.float32) + bi_ref[...])
    t_ref[...] = _softplus(
        jnp.dot(e, wj_ref[...], preferred_element_type=jnp.float32) + bj_ref[...])


def _node_pre(a, Wi, bi, Wj, bj):
    nb = 1000
    grid = N // nb
    blk = pl.BlockSpec((nb, F), lambda i: (i, 0))
    wblk = pl.BlockSpec((F, F), lambda i: (0, 0))
    bblk = pl.BlockSpec((1, F), lambda i: (0, 0))
    out = jax.ShapeDtypeStruct((N, F), jnp.float32)
    return pl.pallas_call(
        _node_pre_body,
        grid=(grid,),
        in_specs=[blk, wblk, bblk, wblk, bblk],
        out_specs=[blk, blk, blk],
        out_shape=[out, out, out],
    )(a, Wi, bi.reshape(1, F), Wj, bj.reshape(1, F))


def _edge_g_body(f_ref, wg_ref, g_ref):
    g_ref[pl.ds(0, EDGES_PER_W), :] = jnp.dot(
        f_ref[...], wg_ref[...], preferred_element_type=jnp.float32)
    g_ref[pl.ds(EDGES_PER_W, RGN - EDGES_PER_W), :] = jnp.zeros(
        (RGN - EDGES_PER_W, F), jnp.float32)


def _edge_g(f_ij, Wg):
    return pl.pallas_call(
        _edge_g_body,
        grid=(NW,),
        in_specs=[pl.BlockSpec((EDGES_PER_W, 16), lambda i: (i, 0)),
                  pl.BlockSpec((16, F), lambda i: (0, 0))],
        out_specs=pl.BlockSpec((RGN, F), lambda i: (i, 0)),
        out_shape=jax.ShapeDtypeStruct((E_PAD, F), jnp.float32),
    )(f_ij, Wg)


def _sc_edges_body(t_hbm, g_hbm, idxi_hbm, idxj_hbm, out_hbm,
                   idxj0, idxj1, idxi0, idxi1, g0, g1, rows0, rows1,
                   semA0, semA1, semB0, semB1, semS0, semS1, acc_sh):
    c = lax.axis_index("c")
    s = lax.axis_index("s")
    wid = c * NS + s
    slots = ((idxj0, idxi0, g0, rows0, semA0, semB0, semS0),
             (idxj1, idxi1, g1, rows1, semA1, semB1, semS1))

    def _zrow(r, _):
        for cc in range(F // 16):
            rows0[r, pl.ds(cc * 16, 16)] = jnp.zeros((16,), jnp.float32)
        return 0
    lax.fori_loop(0, CHUNK, _zrow, 0)

    base_row = s * ROWS_PER_TILE
    for z in range(ROWS_PER_TILE // CHUNK):
        pltpu.sync_copy(rows0, acc_sh.at[pl.ds(base_row + z * CHUNK, CHUNK)])
    plsc.subcore_barrier()

    ebase = wid * RGN

    def startA(ck, sl):
        idxj, idxi, g, _, sA, _, _ = sl
        b = ebase + ck * CHUNK
        pltpu.async_copy(idxj_hbm.at[pl.ds(b, CHUNK)], idxj, sA)
        pltpu.async_copy(idxi_hbm.at[pl.ds(b, CHUNK)], idxi, sA)
        pltpu.async_copy(g_hbm.at[pl.ds(b, CHUNK)], g, sA)

    def waitA(sl):
        idxj, idxi, g, _, sA, _, _ = sl
        pltpu.make_async_copy(idxj_hbm.at[pl.ds(0, CHUNK)], idxj, sA).wait()
        pltpu.make_async_copy(idxi_hbm.at[pl.ds(0, CHUNK)], idxi, sA).wait()
        pltpu.make_async_copy(g_hbm.at[pl.ds(0, CHUNK)], g, sA).wait()

    def startB(sl):
        idxj, _, _, rows, _, sB, _ = sl
        pltpu.async_copy(t_hbm.at[idxj], rows, sB)

    def waitB(sl):
        idxj, _, _, rows, _, sB, _ = sl
        pltpu.make_async_copy(t_hbm.at[idxj], rows, sB).wait()

    def startS(sl):
        _, idxi, _, rows, _, _, sS = sl
        pltpu.async_copy(rows, acc_sh.at[idxi], sS, add=True)

    def waitS(sl):
        _, idxi, _, rows, _, _, sS = sl
        pltpu.make_async_copy(rows, acc_sh.at[idxi], sS).wait()

    def mul(sl):
        _, _, g, rows, _, _, _ = sl
        def _mul(r, _):
            for cc in range(F // 16):
                slc = pl.ds(cc * 16, 16)
                rows[r, slc] = rows[r, slc] * g[r, slc]
            return 0
        lax.fori_loop(0, CHUNK, _mul, 0)

    s0, s1 = slots
    startA(0, s0)
    startA(1, s1)
    waitA(s0)
    startB(s0)

    def pair(p, _):
        cbase = 2 * p
        waitB(s0)
        waitA(s1)
        startB(s1)
        mul(s0)
        startS(s0)
        waitB(s1)
        mul(s1)
        waitS(s0)
        startA(cbase + 2, s0)
        startS(s1)
        waitA(s0)
        startB(s0)
        waitS(s1)
        startA(cbase + 3, s1)
        return 0

    lax.fori_loop(0, PAIRS, pair, 0)
    waitB(s0)
    waitA(s1)
    plsc.subcore_barrier()

    pltpu.sync_copy(acc_sh.at[pl.ds(base_row, ROWS_PER_TILE)],
                    out_hbm.at[c, pl.ds(base_row, ROWS_PER_TILE)])


def _sc_edges(t, g, idx_i_pad, idx_j_pad):
    mesh = plsc.VectorSubcoreMesh(core_axis_name="c", subcore_axis_name="s")
    return pl.kernel(
        _sc_edges_body,
        out_type=jax.ShapeDtypeStruct((NC, N_PAD, F), jnp.float32),
        mesh=mesh,
        scratch_types=[
            pltpu.VMEM((CHUNK,), jnp.int32),
            pltpu.VMEM((CHUNK,), jnp.int32),
            pltpu.VMEM((CHUNK,), jnp.int32),
            pltpu.VMEM((CHUNK,), jnp.int32),
            pltpu.VMEM((CHUNK, F), jnp.float32),
            pltpu.VMEM((CHUNK, F), jnp.float32),
            pltpu.VMEM((CHUNK, F), jnp.float32),
            pltpu.VMEM((CHUNK, F), jnp.float32),
            pltpu.SemaphoreType.DMA,
            pltpu.SemaphoreType.DMA,
            pltpu.SemaphoreType.DMA,
            pltpu.SemaphoreType.DMA,
            pltpu.SemaphoreType.DMA,
            pltpu.SemaphoreType.DMA,
            pltpu.VMEM_SHARED((N_PAD, F), jnp.float32),
        ],
    )(t, g, idx_i_pad, idx_j_pad)


def _post_body(x0_ref, p_ref, emb_ref, rw1_ref, rb1_ref, rw2_ref, rb2_ref,
               gate_ref, wv_ref, bv_ref, ow1_ref, ob1_ref, ow2_ref, ob2_ref,
               wo_ref, bo_ref, eo_ref, pred_ref):
    x = x0_ref[...] + p_ref[0] + p_ref[1]
    for r in range(3):
        h = _softplus(x)
        h = _softplus(jnp.dot(h, rw1_ref[r], preferred_element_type=jnp.float32)
                      + rb1_ref[r])
        h = jnp.dot(h, rw2_ref[r], preferred_element_type=jnp.float32) + rb2_ref[r]
        x = x + h
    x = _softplus(x)
    eo = gate_ref[...] * emb_ref[...] + (
        jnp.dot(x, wv_ref[...], preferred_element_type=jnp.float32) + bv_ref[...])
    eo_ref[...] = eo
    y = eo
    for r in range(2):
        h = _softplus(y)
        h = _softplus(jnp.dot(h, ow1_ref[r], preferred_element_type=jnp.float32)
                      + ob1_ref[r])
        h = jnp.dot(h, ow2_ref[r], preferred_element_type=jnp.float32) + ob2_ref[r]
        y = y + h
    pred_ref[...] = jnp.dot(y, wo_ref[...],
                            preferred_element_type=jnp.float32) + bo_ref[...]


def _post(x0, partials, emb, res_W1, res_b1, res_W2, res_b2, gate, Wv, bv,
          ores_W1, ores_b1, ores_W2, ores_b2, Wo_pad, bo_pad):
    nb = 1000
    grid = N // nb
    blk = pl.BlockSpec((nb, F), lambda i: (i, 0))
    pblk = pl.BlockSpec((NC, nb, F), lambda i: (0, i, 0))
    wfull = pl.BlockSpec((F, F), lambda i: (0, 0))
    brow = pl.BlockSpec((1, F), lambda i: (0, 0))
    w3 = pl.BlockSpec((3, F, F), lambda i: (0, 0, 0))
    b3 = pl.BlockSpec((3, 1, F), lambda i: (0, 0, 0))
    w2 = pl.BlockSpec((2, F, F), lambda i: (0, 0, 0))
    b2 = pl.BlockSpec((2, 1, F), lambda i: (0, 0, 0))
    out = jax.ShapeDtypeStruct((N, F), jnp.float32)
    return pl.pallas_call(
        _post_body,
        grid=(grid,),
        in_specs=[blk, pblk, blk, w3, b3, w3, b3, brow, wfull, brow,
                  w2, b2, w2, b2, wfull, brow],
        out_specs=[blk, blk],
        out_shape=[out, out],
    )(x0, partials, emb,
      res_W1, res_b1.reshape(3, 1, F), res_W2, res_b2.reshape(3, 1, F),
      gate.reshape(1, F), Wv, bv.reshape(1, F),
      ores_W1, ores_b1.reshape(2, 1, F), ores_W2, ores_b2.reshape(2, 1, F),
      Wo_pad, bo_pad)


def kernel(atomic_embedding, f_ij, Wg, Wi, bi, Wj, bj, Wv, bv, gate,
           res_W1, res_b1, res_W2, res_b2,
           ores_W1, ores_b1, ores_W2, ores_b2, Wo, bo, pair_indices):
    emb, x0, t = _node_pre(atomic_embedding, Wi, bi, Wj, bj)
    g = _edge_g(f_ij, Wg)
    ii = jnp.full((NW, RGN), PAD_ROW, jnp.int32)
    ii = ii.at[:, :EDGES_PER_W].set(pair_indices[0].reshape(NW, EDGES_PER_W))
    jj = jnp.zeros((NW, RGN), jnp.int32)
    jj = jj.at[:, :EDGES_PER_W].set(pair_indices[1].reshape(NW, EDGES_PER_W))
    partials = _sc_edges(t, g, ii.reshape(E_PAD), jj.reshape(E_PAD))[:, :N, :]
    Wo_pad = jnp.zeros((F, F), jnp.float32).at[:, :2].set(Wo)
    bo_pad = jnp.zeros((1, F), jnp.float32).at[:, :2].set(bo.reshape(1, 2))
    emb_out, pred_pad = _post(x0, partials, emb,
                              res_W1, res_b1, res_W2, res_b2, gate, Wv, bv,
                              ores_W1, ores_b1, ores_W2, ores_b2, Wo_pad, bo_pad)
    return (pred_pad[:, :2], emb_out)
